# baseline (speedup 1.0000x reference)
import functools

import jax
import jax.numpy as jnp
from jax import lax
from jax.experimental import pallas as pl
from jax.experimental.pallas import tpu as pltpu
from jax.experimental.pallas import tpu_sc as plsc

VOCAB = 1000000
H = 64
B = 16384
L = 15

NC = 2
NS = 16
NW = NC * NS
BPW = B // NW
CB = 32
NCH = BPW // CB
XROWS = CB * L // 120


def _sc_body(center_hbm, ctx_hbm, u_hbm, v_hbm, out_hbm,
             cidx0, cidx1, xidx0, xidx1, ur0, ur1, vr0, vr1, ob0, ob1,
             gsem0, gsem1):
    cid = lax.axis_index("c")
    sid = lax.axis_index("s")
    wid = sid * NC + cid
    base_b = wid * BPW

    lane = lax.iota(jnp.int32, 16)
    lmask = lane < L

    def idx_copy(c, cidx, xidx):
        row0 = pl.multiple_of(base_b + c * CB, CB)
        pltpu.sync_copy(center_hbm.at[pl.ds(row0, CB)], cidx)
        xr0 = pl.multiple_of(wid * (BPW * L // 120) + c * XROWS, XROWS)
        pltpu.sync_copy(ctx_hbm.at[pl.ds(xr0, XROWS)], xidx)

    def gather_descs(cidx, xidx, ur, vr, gsem):
        ds = [pltpu.make_async_copy(u_hbm.at[cidx], ur, gsem)]
        for j in range(XROWS):
            ds.append(pltpu.make_async_copy(
                v_hbm.at[xidx.at[j]], vr.at[pl.ds(j * 120, 120)], gsem))
        return ds

    def compute(ur, vr, ob):
        def bbody(b, carry):
            uvec = [ur[b, pl.ds(t * 16, 16)] for t in range(H // 16)]
            res = jnp.zeros((16,), jnp.float32)
            for l in range(L):
                row = b * L + l
                p = [vr[row, pl.ds(t * 16, 16)] * uvec[t]
                     for t in range(H // 16)]
                d = jnp.sum((p[0] + p[1]) + (p[2] + p[3]))
                res = jnp.where(lane == l, jnp.broadcast_to(d, (16,)), res)
            ob[b, :] = res
            return carry
        lax.fori_loop(0, CB, bbody, 0)

    bufs = ((cidx0, xidx0, ur0, vr0, ob0, gsem0),
            (cidx1, xidx1, ur1, vr1, ob1, gsem1))

    idx_copy(0, bufs[0][0], bufs[0][1])
    for d in gather_descs(bufs[0][0], bufs[0][1], bufs[0][2], bufs[0][3],
                          bufs[0][5]):
        d.start()

    def pair_body(p, carry):
        for par in (0, 1):
            c = p * 2 + par
            cur = bufs[par]
            nxt = bufs[1 - par]
            for d in gather_descs(cur[0], cur[1], cur[2], cur[3], cur[5]):
                d.wait()

            @pl.when(c + 1 < NCH)
            def _prefetch():
                idx_copy(c + 1, nxt[0], nxt[1])
                for d in gather_descs(nxt[0], nxt[1], nxt[2], nxt[3],
                                      nxt[5]):
                    d.start()

            compute(cur[2], cur[3], cur[4])
            row0 = pl.multiple_of(base_b + c * CB, CB)
            pltpu.sync_copy(cur[4], out_hbm.at[pl.ds(row0, CB)])
        return carry

    lax.fori_loop(0, NCH // 2, pair_body, 0)


@jax.jit
def kernel(center_ids, context_neg_ids, U, V):
    center_flat = center_ids.reshape(B).astype(jnp.int32)
    ctx2d = context_neg_ids.reshape(-1, 120).astype(jnp.int32)

    mesh = plsc.VectorSubcoreMesh(core_axis_name="c", subcore_axis_name="s")
    call = functools.partial(
        pl.kernel,
        mesh=mesh,
        compiler_params=pltpu.CompilerParams(needs_layout_passes=False,
                                             use_tc_tiling_on_sc=False),
        out_type=jax.ShapeDtypeStruct((B, 16), jnp.float32),
        scratch_types=[
            pltpu.VMEM((CB,), jnp.int32),
            pltpu.VMEM((CB,), jnp.int32),
            pltpu.VMEM((XROWS, 120), jnp.int32),
            pltpu.VMEM((XROWS, 120), jnp.int32),
            pltpu.VMEM((CB, H), jnp.float32),
            pltpu.VMEM((CB, H), jnp.float32),
            pltpu.VMEM((CB * L, H), jnp.float32),
            pltpu.VMEM((CB * L, H), jnp.float32),
            pltpu.VMEM((CB, 16), jnp.float32),
            pltpu.VMEM((CB, 16), jnp.float32),
            pltpu.SemaphoreType.DMA,
            pltpu.SemaphoreType.DMA,
        ],
    )(_sc_body)
    out_pad = call(center_flat, ctx2d, U, V)
    return out_pad[:, :L]
